# Initial kernel scaffold; baseline (speedup 1.0000x reference)
import jax
import jax.numpy as jnp
from jax.experimental import pallas as pl


def kernel(x, edge_index, edge_attr, batch, atom_emb, bond0, bond1, bond2, W, b, gamma, beta):
    raise NotImplementedError("write your pallas kernel here")


# speedup vs baseline: 2.3706x; 2.3706x over previous
import functools

import jax
import jax.numpy as jnp
from jax.experimental import pallas as pl

N = 10000
E = 320000
H = 128
L = 7
G = 128
EPS = 1e-7
BN = 400


def _post_body(h2_ref, m_ref, w_ref, b_ref, res_ref, out_ref):
    h2 = h2_ref[...]
    m = m_ref[...]
    acc = jnp.dot(h2 + m, w_ref[...], preferred_element_type=jnp.float32)
    out_ref[...] = acc + b_ref[...] + res_ref[...]


def _post(h2, m, W, b, res):
    grid = (N // BN,)
    return pl.pallas_call(
        _post_body,
        grid=grid,
        in_specs=[
            pl.BlockSpec((BN, H), lambda i: (i, 0)),
            pl.BlockSpec((BN, H), lambda i: (i, 0)),
            pl.BlockSpec((H, H), lambda i: (0, 0)),
            pl.BlockSpec((1, H), lambda i: (0, 0)),
            pl.BlockSpec((BN, H), lambda i: (i, 0)),
        ],
        out_specs=pl.BlockSpec((BN, H), lambda i: (i, 0)),
        out_shape=jax.ShapeDtypeStruct((N, H), jnp.float32),
    )(h2, m, W, b.reshape(1, H), res)


def _ln(h, g, b):
    mu = jnp.mean(h, axis=-1, keepdims=True)
    var = jnp.var(h, axis=-1, keepdims=True)
    return (h - mu) / jnp.sqrt(var + 1e-5) * g + b


def kernel(x, edge_index, edge_attr, batch, atom_emb, bond0, bond1, bond2, W, b, gamma, beta):
    src = edge_index[0]
    dst = edge_index[1]
    code = edge_attr[:, 0] * 4 + edge_attr[:, 1] * 2 + edge_attr[:, 2]

    k = jnp.arange(8)
    emb8 = bond0[:, k >> 2, :] + bond1[:, (k >> 1) & 1, :] + bond2[:, k & 1, :]

    base = atom_emb[:, 0, :].sum(axis=0)
    delta = atom_emb[:, 1, :] - atom_emb[:, 0, :]
    xf = x.astype(jnp.float32)
    h0 = base[None, :] + xf @ delta

    def agg(h2, l):
        v = jax.nn.relu(h2[src] + emb8[l][code]) + EPS
        ex = jnp.exp(v)
        denom = jax.ops.segment_sum(ex, dst, num_segments=N)
        numer = jax.ops.segment_sum(v * ex, dst, num_segments=N)
        return numer / (denom + 1e-16)

    zero = jnp.zeros((N, H), jnp.float32)
    h = _post(h0, agg(h0, 0), W[0], b[0], zero)
    for l in range(1, L):
        h2 = jax.nn.relu(_ln(h, gamma[l - 1], beta[l - 1]))
        h = _post(h2, agg(h2, l), W[l], b[l], h)
    h = _ln(h, gamma[L - 1], beta[L - 1])

    ones = jnp.ones((N,), jnp.float32)
    counts = jax.ops.segment_sum(ones, batch, num_segments=G)
    hs = jax.ops.segment_sum(h, batch, num_segments=G)
    return hs / jnp.maximum(counts, 1.0)[:, None]


# speedup vs baseline: 3.3192x; 1.4002x over previous
import functools

import jax
import jax.numpy as jnp
from jax import lax
from jax.experimental import pallas as pl
from jax.experimental.pallas import tpu as pltpu
from jax.experimental.pallas import tpu_sc as plsc

N = 10000
E = 320000
H = 128
L = 7
G = 128
EPS = 1e-7
BN = 400
EP = 327680
RPT = 160
NP = 10240
NPT = NP // 16

_mesh = plsc.VectorSubcoreMesh(core_axis_name="c", subcore_axis_name="s")


@functools.partial(
    pl.kernel,
    out_type=jax.ShapeDtypeStruct((2 * NP, H), jnp.float32),
    mesh=_mesh,
    scratch_types=[
        pltpu.VMEM((8, 128), jnp.int32),
        pltpu.VMEM((8, 128), jnp.int32),
        pltpu.VMEM((128, 128), jnp.float32),
        pltpu.VMEM((128, 128), jnp.float32),
        pltpu.VMEM((32, 128), jnp.float32),
        pltpu.VMEM((32, 128), jnp.float32),
        pltpu.VMEM_SHARED((NP, 128), jnp.float32),
        pltpu.SemaphoreType.DMA,
    ],
)
def _sc_agg(tT, idx2, dst2, m_out, idxb, dstb, gbuf, xybuf, fin, mout, acc, sem):
    c = lax.axis_index("c")
    s = lax.axis_index("s")
    zero16 = jnp.zeros((16,), jnp.float32)

    def zrow(r, _):
        for ch in range(8):
            fin[r, pl.ds(16 * ch, 16)] = zero16
            mout[r, pl.ds(16 * ch, 16)] = zero16
        return 0
    lax.fori_loop(0, 32, zrow, 0)
    for f in range(20):
        pltpu.sync_copy(fin, acc.at[pl.ds(s * NPT + f * 32, 32)])

    plsc.subcore_barrier()

    def block(jb, _):
        pltpu.sync_copy(idx2.at[pl.ds(s * RPT + 8 * jb, 8)], idxb)
        pltpu.sync_copy(dst2.at[pl.ds(s * RPT + 8 * jb, 8)], dstb)
        for u in range(8):
            pltpu.async_copy(tT.at[idxb.at[u]], gbuf, sem).wait()

            def ebody(e, _):
                for ch in range(4):
                    t = gbuf[e, pl.ds(64 * c + 16 * ch, 16)]
                    v = jnp.maximum(t, 0.0) + EPS
                    xv = jnp.exp(v)
                    xybuf[e, pl.ds(16 * ch, 16)] = xv
                    xybuf[e, pl.ds(64 + 16 * ch, 16)] = v * xv
                return 0
            lax.fori_loop(0, 128, ebody, 0)

            pltpu.sync_copy(xybuf, acc.at[dstb.at[u]], add=True)
        return 0
    lax.fori_loop(0, RPT // 8, block, 0)

    plsc.subcore_barrier()

    for f in range(20):
        n0 = s * NPT + f * 32
        pltpu.sync_copy(acc.at[pl.ds(n0, 32)], fin)

        def frow(r, _):
            for ch in range(4):
                x = fin[r, pl.ds(16 * ch, 16)]
                y = fin[r, pl.ds(64 + 16 * ch, 16)]
                mout[r, pl.ds(16 * ch, 16)] = y / (x + 1e-16)
            return 0
        lax.fori_loop(0, 32, frow, 0)
        pltpu.sync_copy(mout, m_out.at[pl.ds(c * NP + n0, 32)])


@functools.partial(
    pl.kernel,
    out_type=jax.ShapeDtypeStruct((2 * G, H), jnp.float32),
    mesh=_mesh,
    scratch_types=[
        pltpu.VMEM((5, 128), jnp.int32),
        pltpu.VMEM((128, 128), jnp.float32),
        pltpu.VMEM((128, 128), jnp.float32),
        pltpu.VMEM((8, 128), jnp.float32),
        pltpu.VMEM((8, 128), jnp.float32),
        pltpu.VMEM_SHARED((G + 8, 128), jnp.float32),
        pltpu.VMEM_SHARED((G + 8, 128), jnp.float32),
    ],
)
def _sc_pool(hT, batch2, out, bt, hv, onesv, fv, cv, accs, accc):
    c = lax.axis_index("c")
    s = lax.axis_index("s")
    zero16 = jnp.zeros((16,), jnp.float32)
    one16 = jnp.full((16,), 1.0, jnp.float32)

    def orow(r, _):
        for ch in range(8):
            onesv[r, pl.ds(16 * ch, 16)] = one16
        return 0
    lax.fori_loop(0, 128, orow, 0)

    def zrow(r, _):
        for ch in range(8):
            fv[r, pl.ds(16 * ch, 16)] = zero16
        return 0
    lax.fori_loop(0, 8, zrow, 0)
    pltpu.sync_copy(fv, accs.at[pl.ds(s * 8, 8)])
    pltpu.sync_copy(fv, accc.at[pl.ds(s * 8, 8)])

    @pl.when(s == 0)
    def _():
        pltpu.sync_copy(fv, accs.at[pl.ds(G, 8)])
        pltpu.sync_copy(fv, accc.at[pl.ds(G, 8)])

    pltpu.sync_copy(batch2.at[s], bt)
    plsc.subcore_barrier()

    for r in range(5):
        pltpu.sync_copy(hT.at[pl.ds(s * NPT + r * 128, 128)], hv)
        pltpu.sync_copy(hv, accs.at[bt.at[r]], add=True)
        pltpu.sync_copy(onesv, accc.at[bt.at[r]], add=True)

    plsc.subcore_barrier()

    pltpu.sync_copy(accs.at[pl.ds(s * 8, 8)], fv)
    pltpu.sync_copy(accc.at[pl.ds(s * 8, 8)], cv)

    def frow(r, _):
        for ch in range(8):
            y = fv[r, pl.ds(16 * ch, 16)]
            n = cv[r, pl.ds(16 * ch, 16)]
            cv[r, pl.ds(16 * ch, 16)] = y / jnp.maximum(n, 1.0)
        return 0
    lax.fori_loop(0, 8, frow, 0)
    pltpu.sync_copy(cv, out.at[pl.ds(c * G + s * 8, 8)])


def _msg_table(t, emb8):
    return t[:, None, :] + emb8[None, :, :]


def _enc_body(xf_ref, d_ref, base_ref, emb_ref, h0_ref, t_ref):
    h0 = jnp.dot(xf_ref[...], d_ref[...], preferred_element_type=jnp.float32)
    h0 = h0 + base_ref[...]
    h0_ref[...] = h0
    t_ref[...] = _msg_table(h0, emb_ref[...])


def _enc(xf, delta, base, emb8):
    return pl.pallas_call(
        _enc_body,
        grid=(N // BN,),
        in_specs=[
            pl.BlockSpec((BN, H), lambda i: (i, 0)),
            pl.BlockSpec((H, H), lambda i: (0, 0)),
            pl.BlockSpec((1, H), lambda i: (0, 0)),
            pl.BlockSpec((8, H), lambda i: (0, 0)),
        ],
        out_specs=[
            pl.BlockSpec((BN, H), lambda i: (i, 0)),
            pl.BlockSpec((BN, 8, H), lambda i: (i, 0, 0)),
        ],
        out_shape=[
            jax.ShapeDtypeStruct((N, H), jnp.float32),
            jax.ShapeDtypeStruct((N, 8, H), jnp.float32),
        ],
    )(xf, delta, base, emb8)


def _ln(h, g, b):
    mu = jnp.mean(h, axis=-1, keepdims=True)
    var = jnp.mean(h * h, axis=-1, keepdims=True) - mu * mu
    return (h - mu) / jnp.sqrt(var + 1e-5) * g + b


def _post_mid_body(h2_ref, m0_ref, m1_ref, w_ref, b_ref, res_ref, g_ref, be_ref,
                   emb_ref, hn_ref, h2n_ref, t_ref, *, first):
    mm = jnp.concatenate([m0_ref[0][:, :64], m1_ref[0][:, :64]], axis=-1)
    hn = jnp.dot(h2_ref[...] + mm, w_ref[...], preferred_element_type=jnp.float32)
    hn = hn + b_ref[...]
    if not first:
        hn = hn + res_ref[...]
    hn_ref[...] = hn
    t2 = jax.nn.relu(_ln(hn, g_ref[...], be_ref[...]))
    h2n_ref[...] = t2
    t_ref[...] = _msg_table(t2, emb_ref[...])


def _post_mid(h2, m, Wl, bl, res, gl, bel, emb8n, first):
    return pl.pallas_call(
        functools.partial(_post_mid_body, first=first),
        grid=(N // BN,),
        in_specs=[
            pl.BlockSpec((BN, H), lambda i: (i, 0)),
            pl.BlockSpec((1, BN, H), lambda i: (0, i, 0)),
            pl.BlockSpec((1, BN, H), lambda i: (1, i, 0)),
            pl.BlockSpec((H, H), lambda i: (0, 0)),
            pl.BlockSpec((1, H), lambda i: (0, 0)),
            pl.BlockSpec((BN, H), lambda i: (i, 0)),
            pl.BlockSpec((1, H), lambda i: (0, 0)),
            pl.BlockSpec((1, H), lambda i: (0, 0)),
            pl.BlockSpec((8, H), lambda i: (0, 0)),
        ],
        out_specs=[
            pl.BlockSpec((BN, H), lambda i: (i, 0)),
            pl.BlockSpec((BN, H), lambda i: (i, 0)),
            pl.BlockSpec((BN, 8, H), lambda i: (i, 0, 0)),
        ],
        out_shape=[
            jax.ShapeDtypeStruct((N, H), jnp.float32),
            jax.ShapeDtypeStruct((N, H), jnp.float32),
            jax.ShapeDtypeStruct((N, 8, H), jnp.float32),
        ],
    )(h2, m, m, Wl, bl, res, gl, bel, emb8n)


def _post_last_body(h2_ref, m0_ref, m1_ref, w_ref, b_ref, res_ref, g_ref, be_ref,
                    hf_ref):
    mm = jnp.concatenate([m0_ref[0][:, :64], m1_ref[0][:, :64]], axis=-1)
    hn = jnp.dot(h2_ref[...] + mm, w_ref[...], preferred_element_type=jnp.float32)
    hn = hn + b_ref[...] + res_ref[...]
    hf_ref[...] = _ln(hn, g_ref[...], be_ref[...])


def _post_last(h2, m, Wl, bl, res, gl, bel):
    return pl.pallas_call(
        _post_last_body,
        grid=(N // BN,),
        in_specs=[
            pl.BlockSpec((BN, H), lambda i: (i, 0)),
            pl.BlockSpec((1, BN, H), lambda i: (0, i, 0)),
            pl.BlockSpec((1, BN, H), lambda i: (1, i, 0)),
            pl.BlockSpec((H, H), lambda i: (0, 0)),
            pl.BlockSpec((1, H), lambda i: (0, 0)),
            pl.BlockSpec((BN, H), lambda i: (i, 0)),
            pl.BlockSpec((1, H), lambda i: (0, 0)),
            pl.BlockSpec((1, H), lambda i: (0, 0)),
        ],
        out_specs=pl.BlockSpec((BN, H), lambda i: (i, 0)),
        out_shape=jax.ShapeDtypeStruct((N, H), jnp.float32),
    )(h2, m, m, Wl, bl, res, gl, bel)


def kernel(x, edge_index, edge_attr, batch, atom_emb, bond0, bond1, bond2, W, b, gamma, beta):
    src = edge_index[0]
    dst = edge_index[1]
    code = edge_attr[:, 0] * 4 + edge_attr[:, 1] * 2 + edge_attr[:, 2]

    idx8 = src * 8 + code
    idx2 = jnp.concatenate([idx8, jnp.zeros((EP - E,), jnp.int32)]).reshape(EP // 128, 128)
    dst2 = jnp.concatenate([dst, jnp.full((EP - E,), NP - 1, jnp.int32)]).reshape(EP // 128, 128)
    batch_pad = jnp.concatenate([batch, jnp.full((NP - N,), G, jnp.int32)])
    batch2 = batch_pad.reshape(16, 5, 128)

    k8 = jnp.arange(8)
    emb8 = bond0[:, k8 >> 2, :] + bond1[:, (k8 >> 1) & 1, :] + bond2[:, k8 & 1, :]

    base = atom_emb[:, 0, :].sum(axis=0).reshape(1, H)
    delta = jnp.zeros((H, H), jnp.float32).at[:9].set(atom_emb[:, 1, :] - atom_emb[:, 0, :])
    xf = jnp.pad(x.astype(jnp.float32), ((0, 0), (0, H - 9)))

    h, T = _enc(xf, delta, base, emb8[0])
    h2 = h
    for l in range(L - 1):
        m = _sc_agg(T.reshape(8 * N, H), idx2, dst2).reshape(2, NP, H)
        h, h2n, T = _post_mid(h2, m, W[l], b[l].reshape(1, H), h,
                              gamma[l].reshape(1, H), beta[l].reshape(1, H),
                              emb8[l + 1], first=(l == 0))
        h2 = h2n
    m = _sc_agg(T.reshape(8 * N, H), idx2, dst2).reshape(2, NP, H)
    hf = _post_last(h2, m, W[L - 1], b[L - 1].reshape(1, H), h,
                    gamma[L - 1].reshape(1, H), beta[L - 1].reshape(1, H))

    hfp = jnp.pad(hf, ((0, NP - N), (0, 0)))
    pool = _sc_pool(hfp, batch2)
    return pool[:G]


# speedup vs baseline: 3.4916x; 1.0520x over previous
import functools

import jax
import jax.numpy as jnp
from jax import lax
from jax.experimental import pallas as pl
from jax.experimental.pallas import tpu as pltpu
from jax.experimental.pallas import tpu_sc as plsc

N = 10000
E = 320000
H = 128
L = 7
G = 128
EPS = 1e-7
BN = 400
EP = 327680
RPT = 160
NP = 10240
NPT = NP // 16

_mesh = plsc.VectorSubcoreMesh(core_axis_name="c", subcore_axis_name="s")


@functools.partial(
    pl.kernel,
    out_type=jax.ShapeDtypeStruct((2 * NP, H), jnp.float32),
    mesh=_mesh,
    scratch_types=[
        pltpu.VMEM((8, 128), jnp.int32),
        pltpu.VMEM((8, 128), jnp.int32),
        pltpu.VMEM((128, 128), jnp.float32),
        pltpu.VMEM((128, 128), jnp.float32),
        pltpu.VMEM((32, 128), jnp.float32),
        pltpu.VMEM((32, 128), jnp.float32),
        pltpu.VMEM_SHARED((NP, 128), jnp.float32),
        pltpu.SemaphoreType.DMA,
        pltpu.SemaphoreType.DMA,
        pltpu.SemaphoreType.DMA,
        pltpu.SemaphoreType.DMA,
    ],
)
def _sc_agg(tT, idx2, dst2, m_out, idxb, dstb, gbuf0, gbuf1, fin, mout, acc,
            gsem0, gsem1, ssem0, ssem1):
    c = lax.axis_index("c")
    s = lax.axis_index("s")
    col0 = 64 * c
    gb = (gbuf0, gbuf1)
    gs = (gsem0, gsem1)
    ss = (ssem0, ssem1)
    zero16 = jnp.zeros((16,), jnp.float32)

    def zrow(r, _):
        for ch in range(8):
            fin[r, pl.ds(16 * ch, 16)] = zero16
            mout[r, pl.ds(16 * ch, 16)] = zero16
        return 0
    lax.fori_loop(0, 32, zrow, 0)
    for f in range(20):
        pltpu.sync_copy(fin, acc.at[pl.ds(s * NPT + f * 32, 32)])

    plsc.subcore_barrier()

    def load_block(jb):
        pltpu.sync_copy(idx2.at[pl.ds(s * RPT + 8 * jb, 8)], idxb)
        pltpu.sync_copy(dst2.at[pl.ds(s * RPT + 8 * jb, 8)], dstb)

    def gather(u, b):
        pltpu.async_copy(tT.at[idxb.at[u]], gb[b], gs[b])

    def gwait(b):
        pltpu.make_async_copy(tT.at[idxb.at[0]], gb[b], gs[b]).wait()

    def scat(u, b):
        pltpu.async_copy(gb[b], acc.at[dstb.at[u]], ss[b], add=True)

    def swait(b):
        pltpu.make_async_copy(gb[b], acc.at[dstb.at[0]], ss[b]).wait()

    def compute(b):
        g = gb[b]

        def ebody(e, _):
            for ch in range(4):
                t = g[e, pl.ds(col0 + 16 * ch, 16)]
                v = jnp.maximum(t, 0.0) + EPS
                xv = jnp.exp(v)
                g[e, pl.ds(16 * ch, 16)] = xv
                g[e, pl.ds(64 + 16 * ch, 16)] = v * xv
            return 0
        lax.fori_loop(0, 128, ebody, 0)

    load_block(0)
    gather(0, 0)
    for u in range(8):
        b = u % 2
        gwait(b)
        compute(b)
        scat(u, b)
        if u == 0:
            gather(1, 1)
        elif u < 7:
            swait(1 - b)
            gather(u + 1, 1 - b)

    def block(jb, _):
        swait(0)
        swait(1)
        load_block(jb)
        gather(0, 0)
        for u in range(8):
            b = u % 2
            gwait(b)
            compute(b)
            scat(u, b)
            if u == 0:
                gather(1, 1)
            elif u < 7:
                swait(1 - b)
                gather(u + 1, 1 - b)
        return 0
    lax.fori_loop(1, RPT // 8, block, 0)
    swait(0)
    swait(1)

    plsc.subcore_barrier()

    for f in range(20):
        n0 = s * NPT + f * 32
        pltpu.sync_copy(acc.at[pl.ds(n0, 32)], fin)

        def frow(r, _):
            for ch in range(4):
                x = fin[r, pl.ds(16 * ch, 16)]
                y = fin[r, pl.ds(64 + 16 * ch, 16)]
                mout[r, pl.ds(16 * ch, 16)] = y / (x + 1e-16)
            return 0
        lax.fori_loop(0, 32, frow, 0)
        pltpu.sync_copy(mout, m_out.at[pl.ds(c * NP + n0, 32)])


@functools.partial(
    pl.kernel,
    out_type=jax.ShapeDtypeStruct((2 * G, H), jnp.float32),
    mesh=_mesh,
    scratch_types=[
        pltpu.VMEM((5, 128), jnp.int32),
        pltpu.VMEM((128, 128), jnp.float32),
        pltpu.VMEM((128, 128), jnp.float32),
        pltpu.VMEM((8, 128), jnp.float32),
        pltpu.VMEM((8, 128), jnp.float32),
        pltpu.VMEM_SHARED((G + 8, 128), jnp.float32),
        pltpu.VMEM_SHARED((G + 8, 128), jnp.float32),
    ],
)
def _sc_pool(hT, batch2, out, bt, hv, onesv, fv, cv, accs, accc):
    c = lax.axis_index("c")
    s = lax.axis_index("s")
    zero16 = jnp.zeros((16,), jnp.float32)
    one16 = jnp.full((16,), 1.0, jnp.float32)

    def orow(r, _):
        for ch in range(8):
            onesv[r, pl.ds(16 * ch, 16)] = one16
        return 0
    lax.fori_loop(0, 128, orow, 0)

    def zrow(r, _):
        for ch in range(8):
            fv[r, pl.ds(16 * ch, 16)] = zero16
        return 0
    lax.fori_loop(0, 8, zrow, 0)
    pltpu.sync_copy(fv, accs.at[pl.ds(s * 8, 8)])
    pltpu.sync_copy(fv, accc.at[pl.ds(s * 8, 8)])

    @pl.when(s == 0)
    def _():
        pltpu.sync_copy(fv, accs.at[pl.ds(G, 8)])
        pltpu.sync_copy(fv, accc.at[pl.ds(G, 8)])

    pltpu.sync_copy(batch2.at[s], bt)
    plsc.subcore_barrier()

    for r in range(5):
        pltpu.sync_copy(hT.at[pl.ds(s * NPT + r * 128, 128)], hv)
        pltpu.sync_copy(hv, accs.at[bt.at[r]], add=True)
        pltpu.sync_copy(onesv, accc.at[bt.at[r]], add=True)

    plsc.subcore_barrier()

    pltpu.sync_copy(accs.at[pl.ds(s * 8, 8)], fv)
    pltpu.sync_copy(accc.at[pl.ds(s * 8, 8)], cv)

    def frow(r, _):
        for ch in range(8):
            y = fv[r, pl.ds(16 * ch, 16)]
            n = cv[r, pl.ds(16 * ch, 16)]
            cv[r, pl.ds(16 * ch, 16)] = y / jnp.maximum(n, 1.0)
        return 0
    lax.fori_loop(0, 8, frow, 0)
    pltpu.sync_copy(cv, out.at[pl.ds(c * G + s * 8, 8)])


def _msg_table(t, emb8):
    return t[:, None, :] + emb8[None, :, :]


def _enc_body(xf_ref, d_ref, base_ref, emb_ref, h0_ref, t_ref):
    h0 = jnp.dot(xf_ref[...], d_ref[...], preferred_element_type=jnp.float32)
    h0 = h0 + base_ref[...]
    h0_ref[...] = h0
    t_ref[...] = _msg_table(h0, emb_ref[...])


def _enc(xf, delta, base, emb8):
    return pl.pallas_call(
        _enc_body,
        grid=(N // BN,),
        in_specs=[
            pl.BlockSpec((BN, H), lambda i: (i, 0)),
            pl.BlockSpec((H, H), lambda i: (0, 0)),
            pl.BlockSpec((1, H), lambda i: (0, 0)),
            pl.BlockSpec((8, H), lambda i: (0, 0)),
        ],
        out_specs=[
            pl.BlockSpec((BN, H), lambda i: (i, 0)),
            pl.BlockSpec((BN, 8, H), lambda i: (i, 0, 0)),
        ],
        out_shape=[
            jax.ShapeDtypeStruct((N, H), jnp.float32),
            jax.ShapeDtypeStruct((N, 8, H), jnp.float32),
        ],
    )(xf, delta, base, emb8)


def _ln(h, g, b):
    mu = jnp.mean(h, axis=-1, keepdims=True)
    var = jnp.mean(h * h, axis=-1, keepdims=True) - mu * mu
    return (h - mu) / jnp.sqrt(var + 1e-5) * g + b


def _post_mid_body(h2_ref, m0_ref, m1_ref, w_ref, b_ref, res_ref, g_ref, be_ref,
                   emb_ref, hn_ref, h2n_ref, t_ref, *, first):
    mm = jnp.concatenate([m0_ref[0][:, :64], m1_ref[0][:, :64]], axis=-1)
    hn = jnp.dot(h2_ref[...] + mm, w_ref[...], preferred_element_type=jnp.float32)
    hn = hn + b_ref[...]
    if not first:
        hn = hn + res_ref[...]
    hn_ref[...] = hn
    t2 = jax.nn.relu(_ln(hn, g_ref[...], be_ref[...]))
    h2n_ref[...] = t2
    t_ref[...] = _msg_table(t2, emb_ref[...])


def _post_mid(h2, m, Wl, bl, res, gl, bel, emb8n, first):
    return pl.pallas_call(
        functools.partial(_post_mid_body, first=first),
        grid=(N // BN,),
        in_specs=[
            pl.BlockSpec((BN, H), lambda i: (i, 0)),
            pl.BlockSpec((1, BN, H), lambda i: (0, i, 0)),
            pl.BlockSpec((1, BN, H), lambda i: (1, i, 0)),
            pl.BlockSpec((H, H), lambda i: (0, 0)),
            pl.BlockSpec((1, H), lambda i: (0, 0)),
            pl.BlockSpec((BN, H), lambda i: (i, 0)),
            pl.BlockSpec((1, H), lambda i: (0, 0)),
            pl.BlockSpec((1, H), lambda i: (0, 0)),
            pl.BlockSpec((8, H), lambda i: (0, 0)),
        ],
        out_specs=[
            pl.BlockSpec((BN, H), lambda i: (i, 0)),
            pl.BlockSpec((BN, H), lambda i: (i, 0)),
            pl.BlockSpec((BN, 8, H), lambda i: (i, 0, 0)),
        ],
        out_shape=[
            jax.ShapeDtypeStruct((N, H), jnp.float32),
            jax.ShapeDtypeStruct((N, H), jnp.float32),
            jax.ShapeDtypeStruct((N, 8, H), jnp.float32),
        ],
    )(h2, m, m, Wl, bl, res, gl, bel, emb8n)


def _post_last_body(h2_ref, m0_ref, m1_ref, w_ref, b_ref, res_ref, g_ref, be_ref,
                    hf_ref):
    mm = jnp.concatenate([m0_ref[0][:, :64], m1_ref[0][:, :64]], axis=-1)
    hn = jnp.dot(h2_ref[...] + mm, w_ref[...], preferred_element_type=jnp.float32)
    hn = hn + b_ref[...] + res_ref[...]
    hf_ref[...] = _ln(hn, g_ref[...], be_ref[...])


def _post_last(h2, m, Wl, bl, res, gl, bel):
    return pl.pallas_call(
        _post_last_body,
        grid=(N // BN,),
        in_specs=[
            pl.BlockSpec((BN, H), lambda i: (i, 0)),
            pl.BlockSpec((1, BN, H), lambda i: (0, i, 0)),
            pl.BlockSpec((1, BN, H), lambda i: (1, i, 0)),
            pl.BlockSpec((H, H), lambda i: (0, 0)),
            pl.BlockSpec((1, H), lambda i: (0, 0)),
            pl.BlockSpec((BN, H), lambda i: (i, 0)),
            pl.BlockSpec((1, H), lambda i: (0, 0)),
            pl.BlockSpec((1, H), lambda i: (0, 0)),
        ],
        out_specs=pl.BlockSpec((BN, H), lambda i: (i, 0)),
        out_shape=jax.ShapeDtypeStruct((N, H), jnp.float32),
    )(h2, m, m, Wl, bl, res, gl, bel)


def kernel(x, edge_index, edge_attr, batch, atom_emb, bond0, bond1, bond2, W, b, gamma, beta):
    src = edge_index[0]
    dst = edge_index[1]
    code = edge_attr[:, 0] * 4 + edge_attr[:, 1] * 2 + edge_attr[:, 2]

    idx8 = src * 8 + code
    idx2 = jnp.concatenate([idx8, jnp.zeros((EP - E,), jnp.int32)]).reshape(EP // 128, 128)
    dst2 = jnp.concatenate([dst, jnp.full((EP - E,), NP - 1, jnp.int32)]).reshape(EP // 128, 128)
    batch_pad = jnp.concatenate([batch, jnp.full((NP - N,), G, jnp.int32)])
    batch2 = batch_pad.reshape(16, 5, 128)

    k8 = jnp.arange(8)
    emb8 = bond0[:, k8 >> 2, :] + bond1[:, (k8 >> 1) & 1, :] + bond2[:, k8 & 1, :]

    base = atom_emb[:, 0, :].sum(axis=0).reshape(1, H)
    delta = jnp.zeros((H, H), jnp.float32).at[:9].set(atom_emb[:, 1, :] - atom_emb[:, 0, :])
    xf = jnp.pad(x.astype(jnp.float32), ((0, 0), (0, H - 9)))

    h, T = _enc(xf, delta, base, emb8[0])
    h2 = h
    for l in range(L - 1):
        m = _sc_agg(T.reshape(8 * N, H), idx2, dst2).reshape(2, NP, H)
        h, h2n, T = _post_mid(h2, m, W[l], b[l].reshape(1, H), h,
                              gamma[l].reshape(1, H), beta[l].reshape(1, H),
                              emb8[l + 1], first=(l == 0))
        h2 = h2n
    m = _sc_agg(T.reshape(8 * N, H), idx2, dst2).reshape(2, NP, H)
    hf = _post_last(h2, m, W[L - 1], b[L - 1].reshape(1, H), h,
                    gamma[L - 1].reshape(1, H), beta[L - 1].reshape(1, H))

    hfp = jnp.pad(hf, ((0, NP - N), (0, 0)))
    pool = _sc_pool(hfp, batch2)
    return pool[:G]
